# baseline (speedup 1.0000x reference)
import functools

import numpy as np

import jax
import jax.numpy as jnp
from jax import lax
from jax.experimental import pallas as pl
from jax.experimental.pallas import tpu as pltpu
from jax.experimental.pallas import tpu_sc as plsc

N = 10000
E = 320000
EMB = 8
RF = 32
CUTOFF = 5.0

NW = 32
BE = 16000
NB = E // BE

NB_HEAD = 3
E_HEAD = NB_HEAD * BE
E_HEAD_PAD = 49152
E_TAIL = E - E_HEAD


@functools.cache
def _sc_hz_kernel(edge_base, stride, chunk, n_out):
    mesh = plsc.VectorSubcoreMesh(core_axis_name="c", subcore_axis_name="s")

    @functools.partial(
        pl.kernel,
        out_type=jax.ShapeDtypeStruct((EMB, n_out), jnp.float32),
        mesh=mesh,
        compiler_params=pltpu.CompilerParams(needs_layout_passes=False),
        scratch_types=[
            pltpu.VMEM((N,), jnp.int32),
            pltpu.VMEM((120 * EMB,), jnp.float32),
            pltpu.VMEM((120 * EMB,), jnp.float32),
            pltpu.VMEM((chunk,), jnp.int32),
            pltpu.VMEM((chunk,), jnp.int32),
            pltpu.VMEM((EMB, chunk), jnp.float32),
        ],
    )
    def _sc_hz(z_hbm, ii_hbm, jj_hbm, ut_hbm, wt_hbm, hz_hbm,
               z_v, ut_v, wt_v, ii_v, jj_v, out_v):
        wid = lax.axis_index("c") * 16 + lax.axis_index("s")
        base = wid * stride
        pltpu.sync_copy(z_hbm, z_v)
        pltpu.sync_copy(ut_hbm, ut_v)
        pltpu.sync_copy(wt_hbm, wt_v)
        pltpu.sync_copy(ii_hbm.at[pl.ds(edge_base + base, chunk)], ii_v)
        pltpu.sync_copy(jj_hbm.at[pl.ds(edge_base + base, chunk)], jj_v)

        def body(s, carry):
            off = s * 16
            ii = ii_v[pl.ds(off, 16)]
            jj = jj_v[pl.ds(off, 16)]
            zi = plsc.load_gather(z_v, [ii]) * EMB
            zj = plsc.load_gather(z_v, [jj]) * EMB
            for c in range(EMB):
                val = plsc.load_gather(ut_v, [zi + c]) \
                    + plsc.load_gather(wt_v, [zj + c])
                out_v[c, pl.ds(off, 16)] = val
            return carry

        lax.fori_loop(0, chunk // 16, body, 0)
        pltpu.sync_copy(out_v, hz_hbm.at[:, pl.ds(base, chunk)])

    return _sc_hz


def _tc_body(r_ref, v_ref, hz_ref, w_ref, b_ref, mus_ref, betas_ref,
             oi_ref, oa_ref, os_ref):
    r = r_ref[...]
    mus = mus_ref[...]
    betas = betas_ref[...]
    d = jnp.exp(-r) - mus
    expansion = jnp.exp(-betas * d * d)
    radial = jnp.dot(w_ref[...], expansion,
                     preferred_element_type=jnp.float32) + b_ref[...]
    env = jnp.where(r < CUTOFF,
                    0.5 * (jnp.cos(r * (np.pi / CUTOFF)) + 1.0), 0.0)
    ehz = env * hz_ref[...]
    c0 = radial[0:8] * ehz
    c1 = radial[8:16] * ehz
    c2 = radial[16:24] * ehz
    v = v_ref[...] / r
    v0, v1, v2 = v[0:1], v[1:2], v[2:3]
    n2_3 = (v0 * v0 + v1 * v1 + v2 * v2) * (1.0 / 3.0)
    zeros = jnp.zeros_like(c0)
    a9 = (None, -v2, v1, v2, None, -v0, -v1, v0, None)
    for g in range(9):
        diag = g % 4 == 0
        oi_ref[8 * g:8 * g + 8, :] = c0 if diag else zeros
        oa_ref[8 * g:8 * g + 8, :] = zeros if a9[g] is None else a9[g] * c1
        s9 = v[g // 3:g // 3 + 1] * v[g % 3:g % 3 + 1]
        os_ref[8 * g:8 * g + 8, :] = (s9 - n2_3 if diag else s9) * c2


def _tc_call(r2, vt, hz, w, b, mus, betas, nb, goff, prev=None):
    whole = lambda shp: pl.BlockSpec(shp, lambda i: (0, 0))
    blk = lambda h: pl.BlockSpec((h, BE), lambda i: (0, i + goff))
    in_specs = [
        blk(1), blk(3), pl.BlockSpec((EMB, BE), lambda i: (0, i)),
        whole((24, RF)), whole((24, 1)), whole((RF, 1)), whole((RF, 1)),
    ]
    inputs = [r2, vt, hz, w, b, mus, betas]
    aliases = {}
    if prev is not None:
        in_specs += [pl.BlockSpec(memory_space=pl.ANY)] * 3
        inputs += list(prev)
        aliases = {7: 0, 8: 1, 9: 2}

    def body(*refs):
        _tc_body(*refs[:7], *refs[-3:])

    return pl.pallas_call(
        body,
        grid=(nb,),
        in_specs=in_specs,
        out_specs=[pl.BlockSpec((72, BE), lambda i: (0, i + goff))] * 3,
        out_shape=[jax.ShapeDtypeStruct((72, E), jnp.float32)] * 3,
        input_output_aliases=aliases,
    )(*inputs)


def kernel(Z, neighbour_index, neighbour_vectors, neighbour_distances,
           z_table, z_map_W, r_map_W, r_map_b, mus, betas):
    ut = z_table @ z_map_W[:, :EMB].T
    wt_tab = z_table @ z_map_W[:, EMB:].T
    ut_p = jnp.zeros((120, EMB), jnp.float32).at[:ut.shape[0]].set(ut).reshape(-1)
    wt_p = jnp.zeros((120, EMB), jnp.float32).at[:ut.shape[0]].set(wt_tab).reshape(-1)
    flat = neighbour_index.astype(jnp.int32).reshape(E, 2)
    zi32 = Z.astype(jnp.int32)
    ii, jj = flat[:, 0], flat[:, 1]
    hz0 = _sc_hz_kernel(0, 1536, 1536, E_HEAD_PAD)(
        zi32, ii, jj, ut_p, wt_p)
    hz1 = _sc_hz_kernel(E_HEAD, 66 * 128, 79 * 128, E_TAIL)(
        zi32, ii, jj, ut_p, wt_p)

    r2 = neighbour_distances.reshape(1, E)
    vt = neighbour_vectors.T
    b = r_map_b.reshape(24, 1)
    mus2, betas2 = mus.reshape(RF, 1), betas.reshape(RF, 1)
    head = _tc_call(r2, vt, hz0, r_map_W, b, mus2, betas2, NB_HEAD, 0)
    oi, oa, osym = _tc_call(r2, vt, hz1, r_map_W, b, mus2, betas2,
                            NB - NB_HEAD, NB_HEAD, prev=head)

    def to_logical(o):
        return o.reshape(3, 3, EMB, E).transpose(3, 2, 0, 1)

    return (to_logical(oi), to_logical(oa), to_logical(osym))


# speedup vs baseline: 1.0255x; 1.0255x over previous
import functools

import numpy as np

import jax
import jax.numpy as jnp
from jax import lax
from jax.experimental import pallas as pl
from jax.experimental.pallas import tpu as pltpu
from jax.experimental.pallas import tpu_sc as plsc

N = 10000
E = 320000
EMB = 8
RF = 32
CUTOFF = 5.0

NW = 32
SC_STRIDE = 9984
SC_CHUNK = 10496
BE = 16000
NB = E // BE


@functools.cache
def _sc_hz_kernel():
    mesh = plsc.VectorSubcoreMesh(core_axis_name="c", subcore_axis_name="s")

    @functools.partial(
        pl.kernel,
        out_type=jax.ShapeDtypeStruct((EMB, E), jnp.float32),
        mesh=mesh,
        compiler_params=pltpu.CompilerParams(needs_layout_passes=False),
        scratch_types=[
            pltpu.VMEM((N,), jnp.int32),
            pltpu.VMEM((120 * EMB,), jnp.float32),
            pltpu.VMEM((120 * EMB,), jnp.float32),
            pltpu.VMEM((SC_CHUNK,), jnp.int32),
            pltpu.VMEM((SC_CHUNK,), jnp.int32),
            pltpu.VMEM((EMB, SC_CHUNK), jnp.float32),
        ],
    )
    def _sc_hz(z_hbm, ii_hbm, jj_hbm, ut_hbm, wt_hbm, hz_hbm,
               z_v, ut_v, wt_v, ii_v, jj_v, out_v):
        wid = lax.axis_index("c") * 16 + lax.axis_index("s")
        base = wid * SC_STRIDE
        pltpu.sync_copy(z_hbm, z_v)
        pltpu.sync_copy(ut_hbm, ut_v)
        pltpu.sync_copy(wt_hbm, wt_v)
        pltpu.sync_copy(ii_hbm.at[pl.ds(base, SC_CHUNK)], ii_v)
        pltpu.sync_copy(jj_hbm.at[pl.ds(base, SC_CHUNK)], jj_v)

        def body(s, carry):
            off = s * 16
            ii = ii_v[pl.ds(off, 16)]
            jj = jj_v[pl.ds(off, 16)]
            zi = plsc.load_gather(z_v, [ii]) * EMB
            zj = plsc.load_gather(z_v, [jj]) * EMB
            for c in range(EMB):
                val = plsc.load_gather(ut_v, [zi + c]) \
                    + plsc.load_gather(wt_v, [zj + c])
                out_v[c, pl.ds(off, 16)] = val
            return carry

        lax.fori_loop(0, SC_CHUNK // 16, body, 0)
        pltpu.sync_copy(out_v, hz_hbm.at[:, pl.ds(base, SC_CHUNK)])

    return _sc_hz


def _tc_body(r_ref, v_ref, hz_ref, w_ref, b_ref, mus_ref, betas_ref,
             oi_ref, oa_ref, os_ref):
    r = r_ref[...]
    mus = mus_ref[...]
    betas = betas_ref[...]
    d = jnp.exp(-r) - mus
    expansion = jnp.exp(-betas * d * d)
    radial = jnp.dot(w_ref[...], expansion,
                     preferred_element_type=jnp.float32) + b_ref[...]
    env = jnp.where(r < CUTOFF,
                    0.5 * (jnp.cos(r * (np.pi / CUTOFF)) + 1.0), 0.0)
    ehz = env * hz_ref[...]
    c0 = radial[0:8] * ehz
    c1 = radial[8:16] * ehz
    c2 = radial[16:24] * ehz
    v = v_ref[...] / r
    v0, v1, v2 = v[0:1], v[1:2], v[2:3]
    n2_3 = (v0 * v0 + v1 * v1 + v2 * v2) * (1.0 / 3.0)
    zeros = jnp.zeros_like(c0)
    a9 = (None, -v2, v1, v2, None, -v0, -v1, v0, None)
    for g in range(9):
        diag = g % 4 == 0
        oi_ref[8 * g:8 * g + 8, :] = c0 if diag else zeros
        oa_ref[8 * g:8 * g + 8, :] = zeros if a9[g] is None else a9[g] * c1
        s9 = v[g // 3:g // 3 + 1] * v[g % 3:g % 3 + 1]
        os_ref[8 * g:8 * g + 8, :] = (s9 - n2_3 if diag else s9) * c2


def _tc_call(r2, vt, hz, w, b, mus, betas):
    whole = lambda shp: pl.BlockSpec(shp, lambda i: (0, 0))
    blk = lambda h: pl.BlockSpec((h, BE), lambda i: (0, i))
    return pl.pallas_call(
        _tc_body,
        grid=(NB,),
        in_specs=[
            blk(1), blk(3), blk(EMB),
            whole((24, RF)), whole((24, 1)), whole((RF, 1)), whole((RF, 1)),
        ],
        out_specs=[pl.BlockSpec((72, BE), lambda i: (0, i))] * 3,
        out_shape=[jax.ShapeDtypeStruct((72, E), jnp.float32)] * 3,
    )(r2, vt, hz, w, b, mus, betas)


def kernel(Z, neighbour_index, neighbour_vectors, neighbour_distances,
           z_table, z_map_W, r_map_W, r_map_b, mus, betas):
    ut = z_table @ z_map_W[:, :EMB].T
    wt_tab = z_table @ z_map_W[:, EMB:].T
    ut_p = jnp.zeros((120, EMB), jnp.float32).at[:ut.shape[0]].set(ut).reshape(-1)
    wt_p = jnp.zeros((120, EMB), jnp.float32).at[:ut.shape[0]].set(wt_tab).reshape(-1)
    flat = neighbour_index.astype(jnp.int32).reshape(E, 2)
    hz = _sc_hz_kernel()(Z.astype(jnp.int32), flat[:, 0], flat[:, 1],
                         ut_p, wt_p)

    r2 = neighbour_distances.reshape(1, E)
    vt = neighbour_vectors.T
    b = r_map_b.reshape(24, 1)
    oi, oa, osym = _tc_call(r2, vt, hz, r_map_W, b,
                            mus.reshape(RF, 1), betas.reshape(RF, 1))

    def to_logical(o):
        return o.reshape(3, 3, EMB, E).transpose(3, 2, 0, 1)

    return (to_logical(oi), to_logical(oa), to_logical(osym))
